# baseline (speedup 1.0000x reference)
import functools

import numpy as np
import jax
import jax.numpy as jnp
from jax import lax
from jax.experimental import pallas as pl
from jax.experimental.pallas import tpu as pltpu
from jax.experimental.pallas import tpu_sc as plsc

EMBED = 768
MAX_LEN = 64
NUM_WORKERS = 32
CHUNK = 64
LANES = 16
EC = EMBED // LANES


def _positional_const():
    pos = np.arange(0, MAX_LEN, dtype=np.float32)[:, None]
    div_term = np.exp(
        np.arange(0, EMBED, 2, dtype=np.float32) * (-np.log(10000.0) / EMBED))
    pe = np.zeros((MAX_LEN, EMBED), dtype=np.float32)
    pe[:, 0::2] = np.sin(pos * div_term)
    pe[:, 1::2] = np.cos(pos * div_term)
    return pe


_PE = _positional_const()


def _make_sc_call(n_tokens):
    per_w = n_tokens // NUM_WORKERS
    n_chunks = per_w // CHUNK
    mesh = plsc.VectorSubcoreMesh(core_axis_name="c", subcore_axis_name="s")

    @functools.partial(
        pl.kernel,
        mesh=mesh,
        out_type=jax.ShapeDtypeStruct((n_tokens, EMBED), jnp.float32),
        scratch_types=[
            pltpu.VMEM((per_w,), jnp.int32),
            pltpu.VMEM((per_w,), jnp.int32),
            pltpu.VMEM((CHUNK, EMBED), jnp.float32),
            pltpu.VMEM((CHUNK, EMBED), jnp.float32),
            pltpu.SemaphoreType.DMA,
            pltpu.SemaphoreType.DMA,
        ],
    )
    def sc_embed(tok_tab_hbm, aux_tab_hbm, tok_idx_hbm, seg_hbm, out_hbm,
                 tok_idx_v, aux_idx_v, tok_buf, aux_buf, sem_tok, sem_aux):
        wid = lax.axis_index("s") * 2 + lax.axis_index("c")
        base = wid * per_w

        pltpu.sync_copy(tok_idx_hbm.at[pl.ds(base, per_w)], tok_idx_v)
        pltpu.sync_copy(seg_hbm.at[pl.ds(base, per_w)], aux_idx_v)

        def mk_idx(i, carry):
            off = pl.multiple_of(i * LANES, 8)
            seg_v = aux_idx_v[pl.ds(off, LANES)]
            pos = i * LANES + lax.iota(jnp.int32, LANES)
            l_v = lax.rem(pos, MAX_LEN)
            aux_idx_v[pl.ds(off, LANES)] = l_v * 3 + seg_v
            return carry

        lax.fori_loop(0, per_w // LANES, mk_idx, 0)

        def chunk_body(g, carry):
            off = pl.multiple_of(g * CHUNK, 8)
            cp_tok = pltpu.async_copy(
                tok_tab_hbm.at[tok_idx_v.at[pl.ds(off, CHUNK)]], tok_buf,
                sem_tok)
            cp_aux = pltpu.async_copy(
                aux_tab_hbm.at[aux_idx_v.at[pl.ds(off, CHUNK)]], aux_buf,
                sem_aux)
            cp_tok.wait()
            cp_aux.wait()

            def add_row(r, c2):
                def add_col(e, c3):
                    col = pl.multiple_of(e * LANES, 8)
                    tok_buf[r, pl.ds(col, LANES)] = (
                        tok_buf[r, pl.ds(col, LANES)]
                        + aux_buf[r, pl.ds(col, LANES)])
                    return c3
                return lax.fori_loop(0, EC, add_col, c2)

            lax.fori_loop(0, CHUNK, add_row, 0)
            pltpu.sync_copy(tok_buf, out_hbm.at[pl.ds(base + off, CHUNK)])
            return carry

        lax.fori_loop(0, n_chunks, chunk_body, 0)

    return sc_embed


def kernel(input, segment_label, token_table, segment_table):
    b, l = input.shape
    n_tokens = b * l
    tok_idx = input.reshape(-1).astype(jnp.int32)
    seg_idx = segment_label.reshape(-1).astype(jnp.int32)
    pe = jnp.asarray(_PE[:l])
    aux_table = (pe[:, None, :] + segment_table[None, :, :].astype(jnp.float32)
                 ).reshape(l * segment_table.shape[0], EMBED)
    out = _make_sc_call(n_tokens)(
        token_table.astype(jnp.float32), aux_table, tok_idx, seg_idx)
    return out.reshape(b, l, EMBED)


# speedup vs baseline: 1.2827x; 1.2827x over previous
import functools

import numpy as np
import jax
import jax.numpy as jnp
from jax import lax
from jax.experimental import pallas as pl
from jax.experimental.pallas import tpu as pltpu
from jax.experimental.pallas import tpu_sc as plsc

EMBED = 768
MAX_LEN = 64
NUM_WORKERS = 32
CHUNK = 32
LANES = 16
EC = EMBED // LANES


def _positional_const():
    pos = np.arange(0, MAX_LEN, dtype=np.float32)[:, None]
    div_term = np.exp(
        np.arange(0, EMBED, 2, dtype=np.float32) * (-np.log(10000.0) / EMBED))
    pe = np.zeros((MAX_LEN, EMBED), dtype=np.float32)
    pe[:, 0::2] = np.sin(pos * div_term)
    pe[:, 1::2] = np.cos(pos * div_term)
    return pe


_PE = _positional_const()


def _make_sc_call(n_tokens):
    per_w = n_tokens // NUM_WORKERS
    n_chunks = per_w // CHUNK
    mesh = plsc.VectorSubcoreMesh(core_axis_name="c", subcore_axis_name="s")

    @functools.partial(
        pl.kernel,
        mesh=mesh,
        out_type=jax.ShapeDtypeStruct((n_tokens, EMBED), jnp.float32),
        scratch_types=[
            pltpu.VMEM((per_w,), jnp.int32),
            pltpu.VMEM((per_w,), jnp.int32),
            pltpu.VMEM((CHUNK, EMBED), jnp.float32),
            pltpu.VMEM((CHUNK, EMBED), jnp.float32),
            pltpu.VMEM((CHUNK, EMBED), jnp.float32),
            pltpu.VMEM((CHUNK, EMBED), jnp.float32),
            pltpu.SemaphoreType.DMA,
            pltpu.SemaphoreType.DMA,
            pltpu.SemaphoreType.DMA,
            pltpu.SemaphoreType.DMA,
        ],
    )
    def sc_embed(tok_tab_hbm, aux_tab_hbm, tok_idx_hbm, seg_hbm, out_hbm,
                 tok_idx_v, aux_idx_v, tok_buf0, tok_buf1, aux_buf0, aux_buf1,
                 sem_t0, sem_t1, sem_a0, sem_a1):
        wid = lax.axis_index("s") * 2 + lax.axis_index("c")
        base = wid * per_w

        pltpu.sync_copy(tok_idx_hbm.at[pl.ds(base, per_w)], tok_idx_v)
        pltpu.sync_copy(seg_hbm.at[pl.ds(base, per_w)], aux_idx_v)

        def mk_idx(i, carry):
            off = pl.multiple_of(i * LANES, 8)
            seg_v = aux_idx_v[pl.ds(off, LANES)]
            pos = i * LANES + lax.iota(jnp.int32, LANES)
            l_v = lax.rem(pos, MAX_LEN)
            aux_idx_v[pl.ds(off, LANES)] = l_v * 3 + seg_v
            return carry

        lax.fori_loop(0, per_w // LANES, mk_idx, 0)

        bufs = ((tok_buf0, aux_buf0, sem_t0, sem_a0),
                (tok_buf1, aux_buf1, sem_t1, sem_a1))

        def gather_copies(off, tb, ab, st, sa):
            return (
                pltpu.make_async_copy(
                    tok_tab_hbm.at[tok_idx_v.at[pl.ds(off, CHUNK)]], tb, st),
                pltpu.make_async_copy(
                    aux_tab_hbm.at[aux_idx_v.at[pl.ds(off, CHUNK)]], ab, sa))

        def gather_start(off, tb, ab, st, sa):
            for cp in gather_copies(off, tb, ab, st, sa):
                cp.start()

        def gather_wait(off, tb, ab, st, sa):
            for cp in gather_copies(off, tb, ab, st, sa):
                cp.wait()

        gather_start(pl.multiple_of(0, 8), *bufs[0])

        def outer(go, carry):
            for b in range(2):
                g = go * 2 + b
                off = pl.multiple_of(g * CHUNK, 8)

                @pl.when(g + 1 < n_chunks)
                def _prefetch():
                    gather_start(pl.multiple_of(off + CHUNK, 8),
                                 *bufs[1 - b])

                gather_wait(off, *bufs[b])
                tb, ab = bufs[b][0], bufs[b][1]

                def add_row(r, c2):
                    def add_col(e, c3):
                        col = pl.multiple_of(e * LANES, 8)
                        tb[r, pl.ds(col, LANES)] = (
                            tb[r, pl.ds(col, LANES)]
                            + ab[r, pl.ds(col, LANES)])
                        return c3
                    return lax.fori_loop(0, EC, add_col, c2)

                lax.fori_loop(0, CHUNK, add_row, 0)
                pltpu.sync_copy(tb, out_hbm.at[pl.ds(base + off, CHUNK)])
            return carry

        lax.fori_loop(0, n_chunks // 2, outer, 0)

    return sc_embed


def kernel(input, segment_label, token_table, segment_table):
    b, l = input.shape
    n_tokens = b * l
    tok_idx = input.reshape(-1).astype(jnp.int32)
    seg_idx = segment_label.reshape(-1).astype(jnp.int32)
    pe = jnp.asarray(_PE[:l])
    aux_table = (pe[:, None, :] + segment_table[None, :, :].astype(jnp.float32)
                 ).reshape(l * segment_table.shape[0], EMBED)
    out = _make_sc_call(n_tokens)(
        token_table.astype(jnp.float32), aux_table, tok_idx, seg_idx)
    return out.reshape(b, l, EMBED)


# speedup vs baseline: 2.3806x; 1.8560x over previous
import functools

import numpy as np
import jax
import jax.numpy as jnp
from jax import lax
from jax.experimental import pallas as pl
from jax.experimental.pallas import tpu as pltpu
from jax.experimental.pallas import tpu_sc as plsc

EMBED = 768
MAX_LEN = 64
NUM_WORKERS = 32
CHUNK = 32
LANES = 16
EC = EMBED // LANES


def _positional_const():
    pos = np.arange(0, MAX_LEN, dtype=np.float32)[:, None]
    div_term = np.exp(
        np.arange(0, EMBED, 2, dtype=np.float32) * (-np.log(10000.0) / EMBED))
    pe = np.zeros((MAX_LEN, EMBED), dtype=np.float32)
    pe[:, 0::2] = np.sin(pos * div_term)
    pe[:, 1::2] = np.cos(pos * div_term)
    return pe


_PE = _positional_const()


def _make_sc_call(n_tokens):
    per_w = n_tokens // NUM_WORKERS
    n_chunks = per_w // CHUNK
    mesh = plsc.VectorSubcoreMesh(core_axis_name="c", subcore_axis_name="s")

    @functools.partial(
        pl.kernel,
        mesh=mesh,
        out_type=jax.ShapeDtypeStruct((n_tokens, EMBED), jnp.float32),
        scratch_types=[
            pltpu.VMEM((per_w,), jnp.int32),
            pltpu.VMEM((per_w,), jnp.int32),
            pltpu.VMEM((CHUNK, EMBED), jnp.float32),
            pltpu.VMEM((CHUNK, EMBED), jnp.float32),
            pltpu.VMEM((CHUNK, EMBED), jnp.float32),
            pltpu.VMEM((CHUNK, EMBED), jnp.float32),
            pltpu.SemaphoreType.DMA,
            pltpu.SemaphoreType.DMA,
            pltpu.SemaphoreType.DMA,
            pltpu.SemaphoreType.DMA,
        ],
    )
    def sc_embed(tok_tab_hbm, aux_tab_hbm, tok_idx_hbm, seg_hbm, out_hbm,
                 tok_idx_v, aux_idx_v, tok_buf0, tok_buf1, aux_buf0, aux_buf1,
                 sem_t0, sem_t1, sem_a0, sem_a1):
        wid = lax.axis_index("s") * 2 + lax.axis_index("c")
        base = wid * per_w

        pltpu.sync_copy(tok_idx_hbm.at[pl.ds(base, per_w)], tok_idx_v)
        pltpu.sync_copy(seg_hbm.at[pl.ds(base, per_w)], aux_idx_v)

        def mk_idx(i, carry):
            off = pl.multiple_of(i * LANES, 8)
            seg_v = aux_idx_v[pl.ds(off, LANES)]
            pos = i * LANES + lax.iota(jnp.int32, LANES)
            l_v = lax.rem(pos, MAX_LEN)
            aux_idx_v[pl.ds(off, LANES)] = l_v * 3 + seg_v
            return carry

        lax.fori_loop(0, per_w // LANES, mk_idx, 0)

        bufs = ((tok_buf0, aux_buf0, sem_t0, sem_a0),
                (tok_buf1, aux_buf1, sem_t1, sem_a1))

        def gather_copies(off, tb, ab, st, sa):
            return (
                pltpu.make_async_copy(
                    tok_tab_hbm.at[tok_idx_v.at[pl.ds(off, CHUNK)]], tb, st),
                pltpu.make_async_copy(
                    aux_tab_hbm.at[aux_idx_v.at[pl.ds(off, CHUNK)]], ab, sa))

        def gather_start(off, tb, ab, st, sa):
            for cp in gather_copies(off, tb, ab, st, sa):
                cp.start()

        def gather_wait(off, tb, ab, st, sa):
            for cp in gather_copies(off, tb, ab, st, sa):
                cp.wait()

        gather_start(pl.multiple_of(0, 8), *bufs[0])

        def outer(go, carry):
            for b in range(2):
                g = go * 2 + b
                off = pl.multiple_of(g * CHUNK, 8)

                @pl.when(g + 1 < n_chunks)
                def _prefetch():
                    gather_start(pl.multiple_of(off + CHUNK, 8),
                                 *bufs[1 - b])

                gather_wait(off, *bufs[b])
                tb, ab = bufs[b][0], bufs[b][1]

                def add_row(r, c2):
                    for e in range(EC):
                        col = e * LANES
                        plsc.addupdate(tb.at[r, pl.ds(col, LANES)],
                                       ab[r, pl.ds(col, LANES)])
                    return c2

                lax.fori_loop(0, CHUNK, add_row, 0)
                pltpu.sync_copy(tb, out_hbm.at[pl.ds(base + off, CHUNK)])
            return carry

        lax.fori_loop(0, n_chunks // 2, outer, 0)

    return sc_embed


def kernel(input, segment_label, token_table, segment_table):
    b, l = input.shape
    n_tokens = b * l
    tok_idx = input.reshape(-1).astype(jnp.int32)
    seg_idx = segment_label.reshape(-1).astype(jnp.int32)
    pe = jnp.asarray(_PE[:l])
    aux_table = (pe[:, None, :] + segment_table[None, :, :].astype(jnp.float32)
                 ).reshape(l * segment_table.shape[0], EMBED)
    out = _make_sc_call(n_tokens)(
        token_table.astype(jnp.float32), aux_table, tok_idx, seg_idx)
    return out.reshape(b, l, EMBED)
